# baseline (speedup 1.0000x reference)
import functools

import jax
import jax.numpy as jnp
from jax import lax
from jax.experimental import pallas as pl
from jax.experimental.pallas import tpu as pltpu
from jax.experimental.pallas import tpu_sc as plsc

_E = 327680
_D = 64
_W = _D // 2
_LANES = 16
_NW = 32
_EPW = _E // _NW
_C = 256
_JPC = _C // 128
_NCHUNK = _EPW // _C


def _sigmoid(x):
    return 1.0 / (1.0 + jnp.exp(-x))


def _sc_body(src_hbm, dst_hbm, u16_hbm, v16_hbm, out_hbm,
             sidx, didx, ubuf0, vbuf0, ubuf1, vbuf1, obuf, sem0, sem1):
    nc = 2
    wid = lax.axis_index("s") * nc + lax.axis_index("c")
    u_hbm = u16_hbm
    v_hbm = v16_hbm

    pltpu.sync_copy(src_hbm.at[pl.ds(wid * _EPW, _EPW)], sidx)
    pltpu.sync_copy(dst_hbm.at[pl.ds(wid * _EPW, _EPW)], didx)

    iota = lax.iota(jnp.int32, _LANES)
    himask = jnp.full((_LANES,), -65536, jnp.int32)

    def fire(g, ubuf, vbuf, sem):
        for j in range(_JPC):
            i0 = g * _C + j * 128
            pltpu.make_async_copy(
                u_hbm.at[sidx.at[pl.ds(i0, 128)]],
                ubuf.at[pl.ds(j * 128, 128)], sem).start()
            pltpu.make_async_copy(
                v_hbm.at[didx.at[pl.ds(i0, 128)]],
                vbuf.at[pl.ds(j * 128, 128)], sem).start()

    def wait(g, ubuf, vbuf, sem):
        for j in range(_JPC):
            i0 = g * _C + j * 128
            pltpu.make_async_copy(
                u_hbm.at[sidx.at[pl.ds(i0, 128)]],
                ubuf.at[pl.ds(j * 128, 128)], sem).wait()
            pltpu.make_async_copy(
                v_hbm.at[didx.at[pl.ds(i0, 128)]],
                vbuf.at[pl.ds(j * 128, 128)], sem).wait()

    def compute(g, ubuf, vbuf):
        out_base = g * _C

        def grp_body(grp, _):
            rows = grp * _LANES + iota

            def w_body(t, acc):
                for k in range(4):
                    w = t * 4 + k
                    cols = (iota + w) & (_W - 1)
                    uw = plsc.load_gather(ubuf, [rows, cols])
                    vw = plsc.load_gather(vbuf, [rows, cols])
                    u0 = plsc.bitcast(uw << 16, jnp.float32)
                    v0 = plsc.bitcast(vw << 16, jnp.float32)
                    u1 = plsc.bitcast(uw & himask, jnp.float32)
                    v1 = plsc.bitcast(vw & himask, jnp.float32)
                    acc = acc + _sigmoid(u0 * v0) + _sigmoid(u1 * v1)
                return acc

            acc = lax.fori_loop(0, _W // 4, w_body,
                                jnp.zeros((_LANES,), jnp.float32))
            obuf[pl.ds(out_base + grp * _LANES, _LANES)] = acc
            return 0

        lax.fori_loop(0, _C // _LANES, grp_body, 0)

    fire(0, ubuf0, vbuf0, sem0)

    def pair_body(gp, _):
        g0 = gp * 2
        fire(g0 + 1, ubuf1, vbuf1, sem1)
        wait(g0, ubuf0, vbuf0, sem0)
        compute(g0, ubuf0, vbuf0)

        @pl.when(g0 + 2 < _NCHUNK)
        def _():
            fire(g0 + 2, ubuf0, vbuf0, sem0)

        wait(g0 + 1, ubuf1, vbuf1, sem1)
        compute(g0 + 1, ubuf1, vbuf1)
        return 0

    lax.fori_loop(0, _NCHUNK // 2, pair_body, 0)

    pltpu.sync_copy(obuf, out_hbm.at[pl.ds(wid * _EPW, _EPW)])


def _sc_call(src, dst, u_packed, v_packed):
    f = pl.kernel(
        _sc_body,
        out_type=jax.ShapeDtypeStruct((_E,), jnp.float32),
        mesh=plsc.VectorSubcoreMesh(core_axis_name="c", subcore_axis_name="s"),
        compiler_params=pltpu.CompilerParams(
            needs_layout_passes=False, use_tc_tiling_on_sc=False),
        scratch_types=[
            pltpu.VMEM((_EPW,), jnp.int32),
            pltpu.VMEM((_EPW,), jnp.int32),
            pltpu.VMEM((_C, _W), jnp.int32),
            pltpu.VMEM((_C, _W), jnp.int32),
            pltpu.VMEM((_C, _W), jnp.int32),
            pltpu.VMEM((_C, _W), jnp.int32),
            pltpu.VMEM((_EPW,), jnp.float32),
            pltpu.SemaphoreType.DMA,
            pltpu.SemaphoreType.DMA,
        ],
    )
    return f(src, dst, u_packed, v_packed)


def _pack(table):
    bits = lax.bitcast_convert_type(table, jnp.int32)
    lo = (bits[:, :_W] >> 16) & jnp.int32(0xFFFF)
    hi = bits[:, _W:] & jnp.int32(-65536)
    return lo | hi


@jax.jit
def kernel(edge_index, U, V):
    src = edge_index[0].astype(jnp.int32)
    dst = edge_index[1].astype(jnp.int32)
    return _sc_call(src, dst, _pack(U), _pack(V))


# speedup vs baseline: 2.5912x; 2.5912x over previous
import functools

import jax
import jax.numpy as jnp
from jax import lax
from jax.experimental import pallas as pl
from jax.experimental.pallas import tpu as pltpu
from jax.experimental.pallas import tpu_sc as plsc

_E = 327680
_D = 64
_LANES = 16
_NW = 32
_EPW = _E // _NW
_C = 256
_IPT = 128
_JPC = _C // _IPT
_NCHUNK = _EPW // _C


def _sigmoid(x):
    return 1.0 / (1.0 + jnp.exp(-x))


def _sc_body(src_hbm, dst_hbm, u_hbm, v_hbm, out_hbm,
             sidx, didx, ubuf0, vbuf0, ubuf1, vbuf1, obuf, sem0, sem1):
    nc = 2
    wid = lax.axis_index("s") * nc + lax.axis_index("c")

    pltpu.sync_copy(src_hbm.at[pl.ds(wid * _EPW, _EPW)], sidx)
    pltpu.sync_copy(dst_hbm.at[pl.ds(wid * _EPW, _EPW)], didx)

    iota = lax.iota(jnp.int32, _LANES)

    def fire(g, ubuf, vbuf, sem):
        for j in range(_JPC):
            i0 = g * _C + j * _IPT
            pltpu.make_async_copy(
                u_hbm.at[sidx.at[pl.ds(i0, _IPT)]],
                ubuf.at[pl.ds(j * _IPT, _IPT)], sem).start()
            pltpu.make_async_copy(
                v_hbm.at[didx.at[pl.ds(i0, _IPT)]],
                vbuf.at[pl.ds(j * _IPT, _IPT)], sem).start()

    def wait(g, ubuf, vbuf, sem):
        for j in range(_JPC):
            i0 = g * _C + j * _IPT
            pltpu.make_async_copy(
                u_hbm.at[sidx.at[pl.ds(i0, _IPT)]],
                ubuf.at[pl.ds(j * _IPT, _IPT)], sem).wait()
            pltpu.make_async_copy(
                v_hbm.at[didx.at[pl.ds(i0, _IPT)]],
                vbuf.at[pl.ds(j * _IPT, _IPT)], sem).wait()

    def compute(g, ubuf, vbuf):
        out_base = g * _C

        def grp_body(grp, _):
            rows = grp * _LANES + iota

            def d_body(t, acc):
                for k in range(4):
                    d = t * 4 + k
                    cols = (iota + d) & (_D - 1)
                    u = plsc.load_gather(ubuf, [rows, cols])
                    v = plsc.load_gather(vbuf, [rows, cols])
                    acc = acc + _sigmoid(u * v)
                return acc

            acc = lax.fori_loop(0, _D // 4, d_body,
                                jnp.zeros((_LANES,), jnp.float32))
            obuf[pl.ds(out_base + grp * _LANES, _LANES)] = acc
            return 0

        lax.fori_loop(0, _C // _LANES, grp_body, 0)

    fire(0, ubuf0, vbuf0, sem0)

    def pair_body(gp, _):
        g0 = gp * 2
        fire(g0 + 1, ubuf1, vbuf1, sem1)
        wait(g0, ubuf0, vbuf0, sem0)
        compute(g0, ubuf0, vbuf0)

        @pl.when(g0 + 2 < _NCHUNK)
        def _():
            fire(g0 + 2, ubuf0, vbuf0, sem0)

        wait(g0 + 1, ubuf1, vbuf1, sem1)
        compute(g0 + 1, ubuf1, vbuf1)
        return 0

    lax.fori_loop(0, _NCHUNK // 2, pair_body, 0)

    pltpu.sync_copy(obuf, out_hbm.at[pl.ds(wid * _EPW, _EPW)])


def _sc_call(src, dst, u, v):
    f = pl.kernel(
        _sc_body,
        out_type=jax.ShapeDtypeStruct((_E,), jnp.float32),
        mesh=plsc.VectorSubcoreMesh(core_axis_name="c", subcore_axis_name="s"),
        compiler_params=pltpu.CompilerParams(
            needs_layout_passes=False, use_tc_tiling_on_sc=False),
        scratch_types=[
            pltpu.VMEM((_EPW,), jnp.int32),
            pltpu.VMEM((_EPW,), jnp.int32),
            pltpu.VMEM((_C, _D), jnp.float32),
            pltpu.VMEM((_C, _D), jnp.float32),
            pltpu.VMEM((_C, _D), jnp.float32),
            pltpu.VMEM((_C, _D), jnp.float32),
            pltpu.VMEM((_EPW,), jnp.float32),
            pltpu.SemaphoreType.DMA,
            pltpu.SemaphoreType.DMA,
        ],
    )
    return f(src, dst, u, v)


@jax.jit
def kernel(edge_index, U, V):
    src = edge_index[0].astype(jnp.int32)
    dst = edge_index[1].astype(jnp.int32)
    return _sc_call(src, dst, U, V)
